# baseline (speedup 1.0000x reference)
import jax
import jax.numpy as jnp
from jax.experimental import pallas as pl
from jax.experimental.pallas import tpu as pltpu

_CB = 128
_NB = 6784

_DN = (((0,), (0,)), ((), ()))


def _fused_heads_body(x_ref, wc_ref, bc_ref, wr_ref, br_ref, wd_ref, bd_ref,
                      cls_ref, reg_ref, dir_ref):
    j = pl.program_id(2)
    xb = x_ref[0].astype(jnp.bfloat16)
    acc_c = jax.lax.dot_general(wc_ref[...], xb, _DN,
                                preferred_element_type=jnp.float32)
    acc_r = jax.lax.dot_general(wr_ref[...], xb, _DN,
                                preferred_element_type=jnp.float32)
    acc_d = jax.lax.dot_general(wd_ref[...], xb, _DN,
                                preferred_element_type=jnp.float32)

    @pl.when(j == 0)
    def _init():
        cls_ref[0] = acc_c + bc_ref[...]
        reg_ref[0] = acc_r + br_ref[...]
        dir_ref[0] = acc_d + bd_ref[...]

    @pl.when(j != 0)
    def _accum():
        cls_ref[0] += acc_c
        reg_ref[0] += acc_r
        dir_ref[0] += acc_d


@jax.jit
def kernel(x, W_cls, b_cls, W_reg, b_reg, W_dir, b_dir):
    B, C, H, W = x.shape
    n = H * W
    xf = x.reshape(B, C, n)

    def _wspec(o):
        return pl.BlockSpec((_CB, o), lambda b, jn, jc: (jc, 0))

    def _bspec(o):
        return pl.BlockSpec((o, 1), lambda b, jn, jc: (0, 0))

    def _ospec(o):
        return pl.BlockSpec((1, o, _NB), lambda b, jn, jc: (b, 0, jn))

    o_cls, o_reg, o_dir = W_cls.shape[0], W_reg.shape[0], W_dir.shape[0]

    cls_f, reg_f, dir_f = pl.pallas_call(
        _fused_heads_body,
        grid=(B, pl.cdiv(n, _NB), C // _CB),
        in_specs=[
            pl.BlockSpec((1, _CB, _NB), lambda b, jn, jc: (b, jc, jn)),
            _wspec(o_cls), _bspec(o_cls),
            _wspec(o_reg), _bspec(o_reg),
            _wspec(o_dir), _bspec(o_dir),
        ],
        compiler_params=pltpu.CompilerParams(
            dimension_semantics=("parallel", "parallel", "arbitrary")),
        out_specs=(_ospec(o_cls), _ospec(o_reg), _ospec(o_dir)),
        out_shape=(
            jax.ShapeDtypeStruct((B, o_cls, n), jnp.float32),
            jax.ShapeDtypeStruct((B, o_reg, n), jnp.float32),
            jax.ShapeDtypeStruct((B, o_dir, n), jnp.float32),
        ),
    )(xf,
      W_cls.T.astype(jnp.bfloat16), b_cls.reshape(o_cls, 1),
      W_reg.T.astype(jnp.bfloat16), b_reg.reshape(o_reg, 1),
      W_dir.T.astype(jnp.bfloat16), b_dir.reshape(o_dir, 1))

    return (cls_f.reshape(B, o_cls, H, W),
            reg_f.reshape(B, o_reg, H, W),
            dir_f.reshape(B, o_dir, H, W))
